# Initial kernel scaffold; baseline (speedup 1.0000x reference)
import jax
import jax.numpy as jnp
from jax.experimental import pallas as pl


def kernel(z, W):
    raise NotImplementedError("write your pallas kernel here")


# speedup vs baseline: 1.5209x; 1.5209x over previous
import jax
import jax.numpy as jnp
from jax.experimental import pallas as pl

N = 18432
K = 1024
D = 64
TN = 512
GRID = N // TN
COMMITMENT_COST = 0.25
LOSS_SCALE = (1.0 + COMMITMENT_COST) / (N * D)


def _vq_body(z_ref, w_ref, zq_ref, loss_ref):
    i = pl.program_id(0)
    z = z_ref[...]
    w = w_ref[...]
    zsq = jnp.sum(z * z, axis=1, keepdims=True)
    wsq = jnp.sum(w * w, axis=1)
    m = jax.lax.dot_general(z, w, (((1,), (1,)), ((), ())),
                            preferred_element_type=jnp.float32)
    d = (zsq + wsq[None, :]) - 2.0 * m
    dmin = jnp.min(d, axis=1, keepdims=True)
    iota = jax.lax.broadcasted_iota(jnp.int32, d.shape, 1)
    idx = jnp.min(jnp.where(d == dmin, iota, K), axis=1)
    one_hot = (iota == idx[:, None]).astype(jnp.float32)
    zq = jax.lax.dot_general(one_hot, w, (((1,), (0,)), ((), ())),
                             preferred_element_type=jnp.float32)
    diff = zq - z
    zq_ref[...] = z + diff
    part = jnp.sum(diff * diff, keepdims=True)

    @pl.when(i == 0)
    def _init():
        loss_ref[...] = jnp.zeros_like(loss_ref)

    loss_ref[...] += part

    @pl.when(i == GRID - 1)
    def _final():
        loss_ref[...] = loss_ref[...] * LOSS_SCALE


def kernel(z, W):
    zq, loss = pl.pallas_call(
        _vq_body,
        grid=(GRID,),
        in_specs=[
            pl.BlockSpec((TN, D), lambda i: (i, 0)),
            pl.BlockSpec((K, D), lambda i: (0, 0)),
        ],
        out_specs=(
            pl.BlockSpec((TN, D), lambda i: (i, 0)),
            pl.BlockSpec((1, 1), lambda i: (0, 0)),
        ),
        out_shape=(
            jax.ShapeDtypeStruct((N, D), jnp.float32),
            jax.ShapeDtypeStruct((1, 1), jnp.float32),
        ),
    )(z, W)
    return zq, loss[0, 0]
